# baseline (speedup 1.0000x reference)
import jax
import jax.numpy as jnp
from jax.experimental import pallas as pl
from jax.experimental.pallas import tpu as pltpu

BS, A, M, T, C, H = 16, 10, 320, 90, 128, 8
B = BS * T
DH = C // H
K1, K2 = 32, 3
NB = 8
R = NB * A * H
F32 = jnp.float32
BF16 = jnp.bfloat16

_INTERPRET = False


def _body(src_ref, tgt_ref, pe1_ref, rel1_ref, pe2_ref, rel2_ref,
          mq_ref, mqf_ref, ex_ref, ext_ref,
          wq_ref, wkv_ref, wp1_ref, wca_ref, wp2_ref, out_ref):
    nt = (((1,), (1,)), ((), ()))
    mq = mq_ref[...]
    mqf = mqf_ref[...]
    ex = ex_ref[...]
    ext = ext_ref[...]

    def tile_rows(x):
        s = x.shape[2]
        return jnp.broadcast_to(x[:, :, None, :], (NB, A, H, s)).reshape(R, s)

    src = src_ref[...].reshape(NB * A, C).astype(BF16)
    q = jnp.dot(src, wq_ref[...], preferred_element_type=F32).astype(BF16)
    qbd = jnp.dot(ex, q, preferred_element_type=F32).astype(BF16) * mq
    tgt = tgt_ref[...].reshape(NB * M, C).astype(BF16)
    kv = jnp.dot(tgt, wkv_ref[...], preferred_element_type=F32).astype(BF16)
    t1 = jnp.transpose(pe1_ref[...].reshape(NB * A, K1, H),
                       (0, 2, 1)).reshape(R, K1)
    pe1 = jnp.take_along_axis(t1, tile_rows(rel1_ref[...]), axis=1)

    def attend(qbd_b, kb, vb, pe_b, mqf_b):
        sc = jax.lax.dot_general(qbd_b, kb, nt, preferred_element_type=F32)
        e = jnp.exp(sc + pe_b)
        recip = 1.0 / jnp.sum(e, axis=-1, keepdims=True)
        o = jnp.dot(e.astype(BF16), vb, preferred_element_type=F32)
        return ((o * recip) * mqf_b).astype(BF16)

    AH = A * H
    os = []
    for b in range(NB):
        os.append(attend(qbd[b * AH:(b + 1) * AH],
                         kv[b * M:(b + 1) * M, :C],
                         kv[b * M:(b + 1) * M, C:],
                         pe1[b * AH:(b + 1) * AH],
                         mqf[b * AH:(b + 1) * AH]))
    om = jnp.concatenate(os, axis=0)
    y = jnp.dot(ext, om, preferred_element_type=F32).astype(BF16)
    y = jnp.dot(y, wp1_ref[...], preferred_element_type=F32).astype(BF16)
    qkv = jnp.dot(y, wca_ref[...], preferred_element_type=F32).astype(BF16)

    q2bd = jnp.dot(ex, qkv[:, :C], preferred_element_type=F32).astype(BF16) * mq
    k2 = qkv[:, C:2 * C]
    v2 = qkv[:, 2 * C:]
    t2 = jnp.transpose(pe2_ref[...].reshape(NB * A, K2, H),
                       (0, 2, 1)).reshape(R, K2)
    pe2 = jnp.take_along_axis(t2, tile_rows(rel2_ref[...]), axis=1)

    os2 = []
    for b in range(NB):
        os2.append(attend(q2bd[b * AH:(b + 1) * AH],
                          k2[b * A:(b + 1) * A],
                          v2[b * A:(b + 1) * A],
                          pe2[b * AH:(b + 1) * AH],
                          mqf[b * AH:(b + 1) * AH]))
    om2 = jnp.concatenate(os2, axis=0)
    z = jnp.dot(ext, om2, preferred_element_type=F32).astype(BF16)
    out = jnp.dot(z, wp2_ref[...], preferred_element_type=F32)
    out_ref[...] = out.reshape(NB, A, C)


def kernel(a_token, m_token, a_pe, a2m_pe, Wq, Wk, Wv, Wp1, Wca, Wp2,
           a_relation, a2m_relation):
    scale = 1.0 / (DH ** 0.5)
    src = a_token.reshape(B, A, C)
    tgt = m_token.reshape(B, M, C)
    wq = (Wq * scale).astype(BF16)
    wkv = jnp.concatenate([Wk, Wv], axis=1).astype(BF16)
    wca = jnp.concatenate([Wca[:, :C] * scale, Wca[:, C:]],
                          axis=1).astype(BF16)
    wp1 = Wp1.astype(BF16)
    wp2 = Wp2.astype(BF16)

    rows = jnp.arange(R, dtype=jnp.int32)
    lanes = jnp.arange(C, dtype=jnp.int32)
    mqf = (lanes[None, :] // DH == rows[:, None] % H).astype(F32)
    mq = mqf.astype(BF16)
    ba = jnp.arange(NB * A, dtype=jnp.int32)
    ex = (rows[:, None] // H == ba[None, :]).astype(BF16)
    ext = (ba[:, None] == rows[None, :] // H).astype(BF16)

    grid = (B // NB,)
    bs = pl.BlockSpec
    out = pl.pallas_call(
        _body,
        grid=grid,
        in_specs=[
            bs((NB, A, C), lambda i: (i, 0, 0)),
            bs((NB, M, C), lambda i: (i, 0, 0)),
            bs((NB * A, K1 * H), lambda i: (i, 0)),
            bs((NB, A, M), lambda i: (i, 0, 0)),
            bs((NB * A, K2 * H), lambda i: (i, 0)),
            bs((NB, A, A), lambda i: (i, 0, 0)),
            bs((R, C), lambda i: (0, 0)),
            bs((R, C), lambda i: (0, 0)),
            bs((R, NB * A), lambda i: (0, 0)),
            bs((NB * A, R), lambda i: (0, 0)),
            bs((C, C), lambda i: (0, 0)),
            bs((C, 2 * C), lambda i: (0, 0)),
            bs((C, C), lambda i: (0, 0)),
            bs((C, 3 * C), lambda i: (0, 0)),
            bs((C, C), lambda i: (0, 0)),
        ],
        out_specs=bs((NB, A, C), lambda i: (i, 0, 0)),
        out_shape=jax.ShapeDtypeStruct((B, A, C), F32),
        compiler_params=pltpu.CompilerParams(
            dimension_semantics=("parallel",)),
        interpret=_INTERPRET,
    )(src, tgt, a2m_pe.reshape(B * A, K1 * H), a2m_relation,
      a_pe.reshape(B * A, K2 * H), a_relation,
      mq, mqf, ex, ext, wq, wkv, wp1, wca, wp2)
    return out


# speedup vs baseline: 1.2580x; 1.2580x over previous
import jax
import jax.numpy as jnp
from jax.experimental import pallas as pl
from jax.experimental.pallas import tpu as pltpu

BS, A, M, T, C, H = 16, 10, 320, 90, 128, 8
B = BS * T
DH = C // H
K1, K2 = 32, 3
NB = 9
NA = NB * A
R = NB * A * H
MB = M // A
F32 = jnp.float32
BF16 = jnp.bfloat16

_INTERPRET = False


def _body(src_ref, tgt_ref, pe1_ref, rel1_ref, pe2_ref, rel2_ref,
          mq_ref, mqf_ref, ex_ref, ext_ref, nm_ref,
          wq_ref, wkv_ref, wp1_ref, wca_ref, wp2_ref, out_ref):
    nt = (((1,), (1,)), ((), ()))
    mq = mq_ref[...]
    mqf = mqf_ref[...]
    ex = ex_ref[...]
    ext = ext_ref[...]
    nm = nm_ref[...]

    def tile_rows(x):
        s = x.shape[2]
        return jnp.broadcast_to(x[:, :, None, :], (NB, A, H, s)).reshape(R, s)

    src = src_ref[...].reshape(NA, C).astype(BF16)
    q = jnp.dot(src, wq_ref[...], preferred_element_type=F32).astype(BF16)
    qbd = jnp.dot(ex, q, preferred_element_type=F32).astype(BF16) * mq
    tgt = tgt_ref[...].reshape(NB * M, C).astype(BF16)
    kv = jnp.dot(tgt, wkv_ref[...], preferred_element_type=F32).astype(BF16)
    t1 = jnp.transpose(pe1_ref[...], (0, 1, 3, 2)).reshape(R, K1)
    pe1 = jnp.take_along_axis(t1, tile_rows(rel1_ref[...]), axis=1)

    def attend(qbd_b, kb, vb, pe_b, mqf_b):
        sc = jax.lax.dot_general(qbd_b, kb, nt, preferred_element_type=F32)
        e = jnp.exp(sc + pe_b)
        recip = 1.0 / jnp.sum(e, axis=-1, keepdims=True)
        o = jnp.dot(e.astype(BF16), vb, preferred_element_type=F32)
        return ((o * recip) * mqf_b).astype(BF16)

    AH = A * H
    os = []
    for b in range(NB):
        os.append(attend(qbd[b * AH:(b + 1) * AH],
                         kv[b * M:(b + 1) * M, :C],
                         kv[b * M:(b + 1) * M, C:],
                         pe1[b * AH:(b + 1) * AH],
                         mqf[b * AH:(b + 1) * AH]))
    om = jnp.concatenate(os, axis=0)
    y = jnp.dot(ext, om, preferred_element_type=F32).astype(BF16)
    y = jnp.dot(y, wp1_ref[...], preferred_element_type=F32).astype(BF16)
    qkv = jnp.dot(y, wca_ref[...], preferred_element_type=F32).astype(BF16)

    q2bd = jnp.dot(ex, qkv[:, :C], preferred_element_type=F32
                   ).astype(BF16) * mq
    k2 = qkv[:, C:2 * C]
    v2 = qkv[:, 2 * C:]
    t2 = jnp.transpose(pe2_ref[...], (0, 1, 3, 2)).reshape(R, K2)
    idx2 = tile_rows(rel2_ref[...])
    idx2w = jnp.broadcast_to(idx2[:, None, :], (R, NB, A)).reshape(R, NA)
    pe2 = jnp.take_along_axis(t2, idx2w, axis=1)

    sc2 = jax.lax.dot_general(q2bd, k2, nt, preferred_element_type=F32)
    e2 = jnp.exp(sc2 + pe2 + nm)
    recip2 = 1.0 / jnp.sum(e2, axis=-1, keepdims=True)
    o2 = jnp.dot(e2.astype(BF16), v2, preferred_element_type=F32)
    om2 = ((o2 * recip2) * mqf).astype(BF16)
    z = jnp.dot(ext, om2, preferred_element_type=F32).astype(BF16)
    out = jnp.dot(z, wp2_ref[...], preferred_element_type=F32)
    out_ref[...] = out.reshape(NB, A, C)


def kernel(a_token, m_token, a_pe, a2m_pe, Wq, Wk, Wv, Wp1, Wca, Wp2,
           a_relation, a2m_relation):
    scale = 1.0 / (DH ** 0.5)
    wq = (Wq * scale).astype(BF16)
    wkv = jnp.concatenate([Wk, Wv], axis=1).astype(BF16)
    wca = jnp.concatenate([Wca[:, :C] * scale, Wca[:, C:]],
                          axis=1).astype(BF16)
    wp1 = Wp1.astype(BF16)
    wp2 = Wp2.astype(BF16)

    rows = jnp.arange(R, dtype=jnp.int32)
    lanes = jnp.arange(C, dtype=jnp.int32)
    mqf = (lanes[None, :] // DH == rows[:, None] % H).astype(F32)
    mq = mqf.astype(BF16)
    ba = jnp.arange(NA, dtype=jnp.int32)
    ex = (rows[:, None] // H == ba[None, :]).astype(BF16)
    ext = (ba[:, None] == rows[None, :] // H).astype(BF16)
    nm = jnp.where(rows[:, None] // (A * H) == ba[None, :] // A,
                   0.0, -1e30).astype(F32)

    grid = (BS, A)
    bs = pl.BlockSpec
    out = pl.pallas_call(
        _body,
        grid=grid,
        in_specs=[
            bs((1, 1, T, C), lambda i, j: (i, j, 0, 0)),
            bs((1, MB, T, C), lambda i, j: (i, j, 0, 0)),
            bs((NB, A, K1, H), lambda i, j: (i * A + j, 0, 0, 0)),
            bs((NB, A, M), lambda i, j: (i * A + j, 0, 0)),
            bs((NB, A, K2, H), lambda i, j: (i * A + j, 0, 0, 0)),
            bs((NB, A, A), lambda i, j: (i * A + j, 0, 0)),
            bs((R, C), lambda i, j: (0, 0)),
            bs((R, C), lambda i, j: (0, 0)),
            bs((R, NA), lambda i, j: (0, 0)),
            bs((NA, R), lambda i, j: (0, 0)),
            bs((R, NA), lambda i, j: (0, 0)),
            bs((C, C), lambda i, j: (0, 0)),
            bs((C, 2 * C), lambda i, j: (0, 0)),
            bs((C, C), lambda i, j: (0, 0)),
            bs((C, 3 * C), lambda i, j: (0, 0)),
            bs((C, C), lambda i, j: (0, 0)),
        ],
        out_specs=bs((NB, A, C), lambda i, j: (i * A + j, 0, 0)),
        out_shape=jax.ShapeDtypeStruct((B, A, C), F32),
        compiler_params=pltpu.CompilerParams(
            dimension_semantics=("parallel", "parallel")),
        interpret=_INTERPRET,
    )(a_token, m_token, a2m_pe, a2m_relation, a_pe, a_relation,
      mq, mqf, ex, ext, nm, wq, wkv, wp1, wca, wp2)
    return out
